# Initial kernel scaffold; baseline (speedup 1.0000x reference)
import functools

import jax
import jax.numpy as jnp
from jax import lax
from jax.experimental import pallas as pl
from jax.experimental.pallas import tpu as pltpu
from jax.experimental.pallas import tpu_sc as plsc

NUM_EMB = 1_000_000
DIM = 32
ROWS = 16384
COLS = 26
B_TOTAL = ROWS * COLS
NC = 2
NS = 16
NW = NC * NS
B_PER_W = B_TOTAL // NW
CHUNK = 128
N_CHUNKS = B_PER_W // CHUNK
NBUF = 4
N_GROUPS = N_CHUNKS // NBUF


def _emb_body(idx_hbm, table_hbm, out_hbm, idx_v, rows_v, *sems):
    gsems = sems[:NBUF]
    wsems = sems[NBUF:]
    wid = lax.axis_index("s") * NC + lax.axis_index("c")
    base = wid * B_PER_W

    pltpu.sync_copy(idx_hbm.at[wid], idx_v)

    def gather(j, b):
        pltpu.make_async_copy(
            table_hbm.at[idx_v.at[j]], rows_v.at[b], gsems[b]
        ).start()

    def gather_wait(j, b):
        pltpu.make_async_copy(
            table_hbm.at[idx_v.at[j]], rows_v.at[b], gsems[b]
        ).wait()

    def write(j, b):
        pltpu.make_async_copy(
            rows_v.at[b], out_hbm.at[pl.ds(base + j * CHUNK, CHUNK)], wsems[b]
        ).start()

    def write_wait(j, b):
        pltpu.make_async_copy(
            rows_v.at[b], out_hbm.at[pl.ds(base + j * CHUNK, CHUNK)], wsems[b]
        ).wait()

    for b in range(NBUF):
        gather(b, b)

    def body(g, carry):
        j0 = g * NBUF
        for b in range(NBUF):
            gather_wait(j0 + b, b)
            write(j0 + b, b)
        jn0 = j0 + NBUF

        @pl.when(g + 1 < N_GROUPS)
        def _():
            for b in range(NBUF):
                write_wait(j0 + b, b)
                gather(jn0 + b, b)

        @pl.when(g + 1 == N_GROUPS)
        def _():
            for b in range(NBUF):
                write_wait(j0 + b, b)

        return carry

    lax.fori_loop(0, N_GROUPS, body, 0)


def kernel(input, weight):
    idx = input.reshape(-1).astype(jnp.int32)
    idx3 = idx.reshape(NW, N_CHUNKS, CHUNK)

    mesh = plsc.VectorSubcoreMesh(core_axis_name="c", subcore_axis_name="s")
    run = pl.kernel(
        _emb_body,
        out_type=jax.ShapeDtypeStruct((B_TOTAL, DIM), jnp.float32),
        mesh=mesh,
        scratch_types=[
            pltpu.VMEM((N_CHUNKS, CHUNK), jnp.int32),
            pltpu.VMEM((NBUF, CHUNK, DIM), jnp.float32),
        ]
        + [pltpu.SemaphoreType.DMA] * (2 * NBUF),
    )
    out = run(idx3, weight)
    return out.reshape(ROWS, COLS, DIM)


# speedup vs baseline: 1.5617x; 1.5617x over previous
import functools

import jax
import jax.numpy as jnp
from jax import lax
from jax.experimental import pallas as pl
from jax.experimental.pallas import tpu as pltpu
from jax.experimental.pallas import tpu_sc as plsc

NUM_EMB = 1_000_000
DIM = 32
ROWS = 16384
COLS = 26
B_TOTAL = ROWS * COLS
NC = 2
NS = 16
NW = NC * NS
B_PER_W = B_TOTAL // NW
CHUNK = 128
N_CHUNKS = B_PER_W // CHUNK
NBUF = 4
N_GROUPS = N_CHUNKS // NBUF


def _emb_body(idx_hbm, table_hbm, out_hbm, idx_v, rows_v, *sems):
    gsems = sems[:NBUF]
    wsems = sems[NBUF:]
    wid = lax.axis_index("s") * NC + lax.axis_index("c")
    base = wid * B_PER_W

    pltpu.sync_copy(idx_hbm.at[wid], idx_v)

    def gather(j, b):
        pltpu.make_async_copy(
            table_hbm.at[idx_v.at[j]], rows_v.at[b], gsems[b]
        ).start()

    def gather_wait(j, b):
        pltpu.make_async_copy(
            table_hbm.at[idx_v.at[j]], rows_v.at[b], gsems[b]
        ).wait()

    def write(j, b):
        pltpu.make_async_copy(
            rows_v.at[b], out_hbm.at[pl.ds(base + j * CHUNK, CHUNK)], wsems[b]
        ).start()

    def write_wait(j, b):
        pltpu.make_async_copy(
            rows_v.at[b], out_hbm.at[pl.ds(base + j * CHUNK, CHUNK)], wsems[b]
        ).wait()

    for b in range(NBUF):
        gather(b, b)

    def body(g, carry):
        j0 = g * NBUF
        for b in range(NBUF):
            gather_wait(j0 + b, b)
            write(j0 + b, b)
        jn0 = j0 + NBUF

        @pl.when(g + 1 < N_GROUPS)
        def _():
            for b in range(NBUF):
                write_wait(j0 + b, b)
                gather(jn0 + b, b)

        @pl.when(g + 1 == N_GROUPS)
        def _():
            for b in range(NBUF):
                write_wait(j0 + b, b)

        return carry

    lax.fori_loop(0, N_GROUPS, body, 0)


def kernel(input, weight):
    idx = input.reshape(-1).astype(jnp.int32)
    idx3 = idx.reshape(NW, N_CHUNKS, CHUNK)

    mesh = plsc.VectorSubcoreMesh(core_axis_name="c", subcore_axis_name="s")
    run = pl.kernel(
        _emb_body,
        out_type=jax.ShapeDtypeStruct((B_TOTAL, DIM), jnp.float32),
        mesh=mesh,
        scratch_types=[
            pltpu.VMEM((N_CHUNKS, CHUNK), jnp.int32),
            pltpu.VMEM((NBUF, CHUNK, DIM), jnp.float32),
        ]
        + [pltpu.SemaphoreType.DMA] * (2 * NBUF),
        compiler_params=pltpu.CompilerParams(use_tc_tiling_on_sc=False),
    )
    out = run(idx3, weight)
    return out.reshape(ROWS, COLS, DIM)


# speedup vs baseline: 1.5780x; 1.0104x over previous
import functools

import jax
import jax.numpy as jnp
from jax import lax
from jax.experimental import pallas as pl
from jax.experimental.pallas import tpu as pltpu
from jax.experimental.pallas import tpu_sc as plsc

NUM_EMB = 1_000_000
DIM = 32
ROWS = 16384
COLS = 26
B_TOTAL = ROWS * COLS
NC = 2
NS = 16
NW = NC * NS
B_PER_W = B_TOTAL // NW
CHUNK = 128
N_CHUNKS = B_PER_W // CHUNK
NBUF = 8
N_GROUPS = N_CHUNKS // NBUF


def _emb_body(idx_hbm, table_hbm, out_hbm, idx_v, rows_v, *sems):
    gsems = sems[:NBUF]
    wsems = sems[NBUF:]
    wid = lax.axis_index("s") * NC + lax.axis_index("c")
    base = wid * B_PER_W

    pltpu.sync_copy(idx_hbm.at[wid], idx_v)

    def gather(j, b):
        pltpu.make_async_copy(
            table_hbm.at[idx_v.at[j]], rows_v.at[b], gsems[b]
        ).start()

    def gather_wait(j, b):
        pltpu.make_async_copy(
            table_hbm.at[idx_v.at[j]], rows_v.at[b], gsems[b]
        ).wait()

    def write(j, b):
        pltpu.make_async_copy(
            rows_v.at[b], out_hbm.at[pl.ds(base + j * CHUNK, CHUNK)], wsems[b]
        ).start()

    def write_wait(j, b):
        pltpu.make_async_copy(
            rows_v.at[b], out_hbm.at[pl.ds(base + j * CHUNK, CHUNK)], wsems[b]
        ).wait()

    for b in range(NBUF - 1):
        gather(b, b)

    def body(g, carry):
        j0 = g * NBUF
        for b in range(NBUF):
            j = j0 + b
            gather_wait(j, b)
            write(j, b)
            bp = (b - 1) % NBUF
            jn = j + NBUF - 1

            if b == 0:
                @pl.when(j >= 1)
                def _():
                    write_wait(j - 1, bp)

                gather(jn, bp)
            else:
                @pl.when(jn < N_CHUNKS)
                def _():
                    write_wait(j - 1, bp)
                    gather(jn, bp)

        return carry

    lax.fori_loop(0, N_GROUPS, body, 0)

    for b in range(NBUF):
        write_wait(N_CHUNKS - NBUF + b, b)


def kernel(input, weight):
    idx = input.reshape(-1).astype(jnp.int32)
    idx3 = idx.reshape(NW, N_CHUNKS, CHUNK)

    mesh = plsc.VectorSubcoreMesh(core_axis_name="c", subcore_axis_name="s")
    run = pl.kernel(
        _emb_body,
        out_type=jax.ShapeDtypeStruct((B_TOTAL, DIM), jnp.float32),
        mesh=mesh,
        scratch_types=[
            pltpu.VMEM((N_CHUNKS, CHUNK), jnp.int32),
            pltpu.VMEM((NBUF, CHUNK, DIM), jnp.float32),
        ]
        + [pltpu.SemaphoreType.DMA] * (2 * NBUF),
        compiler_params=pltpu.CompilerParams(use_tc_tiling_on_sc=False),
    )
    out = run(idx3, weight)
    return out.reshape(ROWS, COLS, DIM)
